# Initial kernel scaffold; baseline (speedup 1.0000x reference)
import jax
import jax.numpy as jnp
from jax.experimental import pallas as pl


def kernel(hidden_states, W_route, b_route):
    raise NotImplementedError("write your pallas kernel here")


# speedup vs baseline: 1.1447x; 1.1447x over previous
import functools

import jax
import jax.numpy as jnp
from jax import lax
from jax.experimental import pallas as pl
from jax.experimental.pallas import tpu as pltpu

_ROUTE_FRAC = 0.15
_ENTROPY_WEIGHT = 0.4
_COHERENCE_WEIGHT = 0.4
_LEARNED_WEIGHT = 0.2
_WINDOW = 8


def _moving_avg(x, window):
    pad_l = window // 2
    pad_r = window - 1 - pad_l
    xp = jnp.pad(x, ((0, 0), (pad_l, pad_r)), mode='edge')
    cs = jnp.cumsum(xp, axis=1)
    cs = jnp.pad(cs, ((0, 0), (1, 0)))
    return (cs[:, window:] - cs[:, :-window]) / window


def _routing_scores(hidden_states, W_route, b_route):
    variance = jnp.var(hidden_states, axis=-1, ddof=1)
    entropy_score = jax.nn.sigmoid(variance)
    scores = _ENTROPY_WEIGHT * entropy_score
    c = jnp.mean(jnp.cos(hidden_states), axis=-1)
    s = jnp.mean(jnp.sin(hidden_states), axis=-1)
    token_coh = jnp.sqrt(c * c + s * s + 1e-12)
    ca = _moving_avg(c, _WINDOW)
    sa = _moving_avg(s, _WINDOW)
    local_coh = jnp.sqrt(ca * ca + sa * sa + 1e-12)
    coherence = 0.5 * token_coh + 0.5 * local_coh
    scores = scores + _COHERENCE_WEIGHT * coherence
    learned = jax.nn.sigmoid(
        jnp.squeeze(hidden_states @ W_route + b_route, axis=-1))
    scores = scores + _LEARNED_WEIGHT * learned
    return scores


def _select_body(scores_ref, attn_ref, mix_ref, *, k):
    sc = scores_ref[...]
    bdim, ldim = sc.shape
    ib = lax.bitcast_convert_type(sc, jnp.int32)
    minint = jnp.int32(-(2 ** 31))
    kk = jnp.where(ib < 0, jnp.bitwise_not(ib), jnp.bitwise_xor(ib, minint))

    one = jnp.int32(1)
    mtwo = jnp.int32(-2)

    def radix_step(i, carry):
        p, r = carry
        bit = 31 - i
        himask = jnp.left_shift(mtwo, bit)
        match = jnp.bitwise_and(kk, himask) == jnp.bitwise_and(p, himask)
        bits0 = jnp.bitwise_and(jnp.right_shift(kk, bit), one) == 0
        c0 = jnp.sum((match & bits0).astype(jnp.int32), axis=1, keepdims=True)
        take0 = r <= c0
        p = jnp.where(take0, p, jnp.bitwise_or(p, jnp.left_shift(one, bit)))
        r = jnp.where(take0, r, r - c0)
        return p, r

    p0 = jnp.zeros((bdim, 1), jnp.int32)
    r0 = jnp.full((bdim, 1), k, jnp.int32)
    p, r = lax.fori_loop(0, 32, radix_step, (p0, r0))

    eq = kk == p
    less = jnp.bitwise_xor(kk, minint) < jnp.bitwise_xor(p, minint)

    idx = lax.broadcasted_iota(jnp.int32, (bdim, ldim), 1)
    nbits = max(1, (ldim - 1).bit_length())

    def idx_step(i, carry):
        q, r = carry
        bit = nbits - 1 - i
        himask = jnp.left_shift(mtwo, bit)
        match = (kk == p) & (
            jnp.bitwise_and(idx, himask) == jnp.bitwise_and(q, himask))
        bits0 = jnp.bitwise_and(jnp.right_shift(idx, bit), one) == 0
        c0 = jnp.sum((match & bits0).astype(jnp.int32), axis=1, keepdims=True)
        take0 = r <= c0
        q = jnp.where(take0, q, jnp.bitwise_or(q, jnp.left_shift(one, bit)))
        r = jnp.where(take0, r, r - c0)
        return q, r

    q0 = jnp.zeros((bdim, 1), jnp.int32)
    q, _ = lax.fori_loop(0, nbits, idx_step, (q0, r))

    attn = less | (eq & (idx <= q))
    attn_ref[...] = attn.astype(jnp.int32)
    mix_ref[...] = jnp.logical_not(attn).astype(jnp.int32)


def kernel(hidden_states, W_route, b_route):
    b, l, d = hidden_states.shape
    scores = _routing_scores(hidden_states, W_route, b_route)
    n_attn = max(1, int(l * _ROUTE_FRAC))
    attn_i, mix_i = pl.pallas_call(
        functools.partial(_select_body, k=n_attn),
        out_shape=[jax.ShapeDtypeStruct((b, l), jnp.int32)] * 2,
    )(scores)
    attn_mask = attn_i.astype(bool)
    mix_mask = mix_i.astype(bool)
    return (attn_mask, mix_mask, scores)
